# Initial kernel scaffold; baseline (speedup 1.0000x reference)
import functools

import jax
import jax.numpy as jnp
from jax import lax
from jax.experimental import pallas as pl
from jax.experimental.pallas import tpu as pltpu
from jax.experimental.pallas import tpu_sc as plsc

N = 100000
E = 1600000
EMB = 32
G = 64
HALF = 16
NSUB = 16
RPT = 800
ROWS = NSUB * RPT
E2 = ROWS * 128
CHUNK = 16
NCHUNKS = RPT // CHUNK
ACC_ROWS = N + 16
ZROWS = 1250
RB = 1000


def _seg_body(src_hbm, dst_hbm, y2_hbm, out_hbm, sidx, didx, rows, zbuf,
              acc, semg, sems):
    c = lax.axis_index("c")
    s = lax.axis_index("s")

    def zb(i, carry):
        zbuf[i, :] = jnp.zeros((16,), jnp.float32)
        return carry
    lax.fori_loop(0, ZROWS, zb, 0)
    zbase = s * (5 * ZROWS)
    for k in range(5):
        pltpu.sync_copy(zbuf, acc.at[pl.ds(zbase + k * ZROWS, ZROWS)])
    plsc.subcore_barrier()

    tilebase = s * RPT

    def chunk(i, carry):
        r0 = tilebase + i * CHUNK
        pltpu.sync_copy(src_hbm.at[c, pl.ds(r0, CHUNK)], sidx)
        pltpu.sync_copy(dst_hbm.at[pl.ds(r0, CHUNK)], didx)
        gathers = [pltpu.async_copy(y2_hbm.at[sidx.at[j]], rows.at[j], semg)
                   for j in range(CHUNK)]
        for gcp in gathers:
            gcp.wait()
        scats = [pltpu.async_copy(rows.at[j], acc.at[didx.at[j]], sems,
                                  add=True)
                 for j in range(CHUNK)]
        for scp in scats:
            scp.wait()
        return carry
    lax.fori_loop(0, NCHUNKS, chunk, 0)
    plsc.subcore_barrier()

    ob = s * (N // NSUB)
    pltpu.sync_copy(acc.at[pl.ds(ob, N // NSUB)],
                    out_hbm.at[c, pl.ds(ob, N // NSUB)])


_segment_sum_sc = functools.partial(
    pl.kernel,
    out_type=jax.ShapeDtypeStruct((2, N, HALF), jnp.float32),
    mesh=plsc.VectorSubcoreMesh(core_axis_name="c", subcore_axis_name="s"),
    scratch_types=[
        pltpu.VMEM((CHUNK, 128), jnp.int32),
        pltpu.VMEM((CHUNK, 128), jnp.int32),
        pltpu.VMEM((CHUNK, 128, HALF), jnp.float32),
        pltpu.VMEM((ZROWS, HALF), jnp.float32),
        pltpu.VMEM_SHARED((ACC_ROWS, HALF), jnp.float32),
        pltpu.SemaphoreType.DMA,
        pltpu.SemaphoreType.DMA,
    ],
)(_seg_body)


def _segment_sum(src_ab, dst_r, y):
    return _segment_sum_sc(src_ab, dst_r, y.reshape(2 * N, HALF))


def _mm_body(x_ref, w_ref, o_ref):
    o_ref[...] = jnp.dot(x_ref[...], w_ref[...],
                         preferred_element_type=jnp.float32)


def _proj(x, w):
    din = x.shape[1]
    return pl.pallas_call(
        _mm_body,
        grid=(N // RB,),
        in_specs=[pl.BlockSpec((RB, din), lambda i: (i, 0)),
                  pl.BlockSpec((din, EMB), lambda i: (0, 0))],
        out_specs=pl.BlockSpec((RB, EMB), lambda i: (i, 0)),
        out_shape=jax.ShapeDtypeStruct((N, EMB), jnp.float32),
    )(x, w)


def _mlp_body(y_ref, agg_ref, b1_ref, w2_ref, b2_ref, t_ref, st_ref):
    aggf = jnp.concatenate([agg_ref[0], agg_ref[1]], axis=1)
    h = jnp.maximum(y_ref[...] + aggf + b1_ref[0:1, :], 0.0)
    t = jnp.dot(h, w2_ref[...], preferred_element_type=jnp.float32) \
        + b2_ref[0:1, :]
    t_ref[...] = t

    @pl.when(pl.program_id(0) == 0)
    def _():
        st_ref[...] = jnp.zeros_like(st_ref)
    s1 = jnp.sum(t, axis=0, keepdims=True)
    s2 = jnp.sum(t * t, axis=0, keepdims=True)
    st_ref[...] += jnp.concatenate(
        [s1, s2, jnp.zeros((6, EMB), jnp.float32)], axis=0)


def _mlp(y, agg, b1, w2, b2):
    return pl.pallas_call(
        _mlp_body,
        grid=(N // RB,),
        in_specs=[pl.BlockSpec((RB, EMB), lambda i: (i, 0)),
                  pl.BlockSpec((2, RB, HALF), lambda i: (0, i, 0)),
                  pl.BlockSpec((8, EMB), lambda i: (0, 0)),
                  pl.BlockSpec((EMB, EMB), lambda i: (0, 0)),
                  pl.BlockSpec((8, EMB), lambda i: (0, 0))],
        out_specs=[pl.BlockSpec((RB, EMB), lambda i: (i, 0)),
                   pl.BlockSpec((8, EMB), lambda i: (0, 0))],
        out_shape=[jax.ShapeDtypeStruct((N, EMB), jnp.float32),
                   jax.ShapeDtypeStruct((8, EMB), jnp.float32)],
    )(y, agg, b1, w2, b2)


def _norm_proj_body(t_ref, sc_ref, sh_ref, w_ref, o_ref):
    z = jnp.maximum(t_ref[...] * sc_ref[0:1, :] + sh_ref[0:1, :], 0.0)
    o_ref[...] = jnp.dot(z, w_ref[...], preferred_element_type=jnp.float32)


def _norm_proj(t, scale, shift, w_next):
    return pl.pallas_call(
        _norm_proj_body,
        grid=(N // RB,),
        in_specs=[pl.BlockSpec((RB, EMB), lambda i: (i, 0)),
                  pl.BlockSpec((8, EMB), lambda i: (0, 0)),
                  pl.BlockSpec((8, EMB), lambda i: (0, 0)),
                  pl.BlockSpec((EMB, EMB), lambda i: (0, 0))],
        out_specs=pl.BlockSpec((RB, EMB), lambda i: (i, 0)),
        out_shape=jax.ShapeDtypeStruct((N, EMB), jnp.float32),
    )(t, scale, shift, w_next)


def _norm_pool_body(t_ref, sc_ref, sh_ref, bat_ref, z_ref, gs_ref, gc_ref):
    z = jnp.maximum(t_ref[...] * sc_ref[0:1, :] + sh_ref[0:1, :], 0.0)
    z_ref[...] = z
    labels = bat_ref[0, 0, :]
    gids = lax.broadcasted_iota(jnp.int32, (RB, G), 1)
    oh = (labels[:, None] == gids).astype(jnp.float32)

    @pl.when(pl.program_id(0) == 0)
    def _():
        gs_ref[...] = jnp.zeros_like(gs_ref)
        gc_ref[...] = jnp.zeros_like(gc_ref)
    gs_ref[...] += lax.dot_general(oh, z, (((0,), (0,)), ((), ())),
                                   preferred_element_type=jnp.float32)
    cnt = jnp.sum(oh, axis=0, keepdims=True)
    gc_ref[...] += jnp.concatenate(
        [cnt, jnp.zeros((7, G), jnp.float32)], axis=0)


def _norm_pool(t, scale, shift, batch3):
    return pl.pallas_call(
        _norm_pool_body,
        grid=(N // RB,),
        in_specs=[pl.BlockSpec((RB, EMB), lambda i: (i, 0)),
                  pl.BlockSpec((8, EMB), lambda i: (0, 0)),
                  pl.BlockSpec((8, EMB), lambda i: (0, 0)),
                  pl.BlockSpec((1, 1, RB), lambda i: (i, 0, 0))],
        out_specs=[pl.BlockSpec((RB, EMB), lambda i: (i, 0)),
                   pl.BlockSpec((G, EMB), lambda i: (0, 0)),
                   pl.BlockSpec((8, G), lambda i: (0, 0))],
        out_shape=[jax.ShapeDtypeStruct((N, EMB), jnp.float32),
                   jax.ShapeDtypeStruct((G, EMB), jnp.float32),
                   jax.ShapeDtypeStruct((8, G), jnp.float32)],
    )(t, scale, shift, batch3)


def _bn_coeffs(stats, gamma, beta):
    mean = stats[0] / N
    var = stats[1] / N - mean * mean
    rstd = lax.rsqrt(var + 1e-5)
    scale = gamma * rstd
    shift = beta - mean * scale
    return (jnp.broadcast_to(scale[None, :], (8, EMB)),
            jnp.broadcast_to(shift[None, :], (8, EMB)))


def kernel(x, edge_index, batch,
           W1_0, b1_0, W2_0, b2_0, gamma_0, beta_0,
           W1_1, b1_1, W2_1, b2_1, gamma_1, beta_1,
           W1_2, b1_2, W2_2, b2_2, gamma_2, beta_2):
    src = edge_index[0]
    dst = edge_index[1]
    pad = E2 - E
    src2 = src * 2
    zpad = jnp.zeros((pad,), jnp.int32)
    src_ab = jnp.stack([
        jnp.concatenate([src2, zpad]),
        jnp.concatenate([src2 + 1, zpad]),
    ]).reshape(2, ROWS, 128)
    dst_r = jnp.concatenate([dst, jnp.full((pad,), N, jnp.int32)]) \
        .reshape(ROWS, 128)
    batch3 = batch.reshape(N // RB, 1, RB)

    b1s = [b1_0, b1_1, b1_2]
    w2s = [W2_0, W2_1, W2_2]
    b2s = [b2_0, b2_1, b2_2]
    gammas = [gamma_0, gamma_1, gamma_2]
    betas = [beta_0, beta_1, beta_2]
    w1_next = [W1_1, W1_2]

    y = _proj(x, W1_0)
    node_rep = None
    for i in range(3):
        agg = _segment_sum(src_ab, dst_r, y)
        b18 = jnp.broadcast_to(b1s[i][None, :], (8, EMB))
        b28 = jnp.broadcast_to(b2s[i][None, :], (8, EMB))
        t, stats = _mlp(y, agg, b18, w2s[i], b28)
        scale8, shift8 = _bn_coeffs(stats, gammas[i], betas[i])
        if i < 2:
            y = _norm_proj(t, scale8, shift8, w1_next[i])
        else:
            node_rep, gsum, gcnt = _norm_pool(t, scale8, shift8, batch3)
    counts = gcnt[0]
    graph_rep = gsum / jnp.maximum(counts, 1.0)[:, None]
    return node_rep, graph_rep


# speedup vs baseline: 6.8295x; 6.8295x over previous
import functools

import jax
import jax.numpy as jnp
from jax import lax
from jax.experimental import pallas as pl
from jax.experimental.pallas import tpu as pltpu
from jax.experimental.pallas import tpu_sc as plsc

N = 100000
E = 1600000
EMB = 32
G = 64
HALF = 16
NSUB = 16
RPT = 800
ROWS = NSUB * RPT
E2 = ROWS * 128
CHUNK = 8
NCHUNKS = RPT // CHUNK
ACC_ROWS = 100096
STRIPE = ACC_ROWS // NSUB
NZCOPY = 16
ZROWS = STRIPE // NZCOPY
RB = 1000


def _make_seg_body(h):
    def _seg_body(src_hbm, dst_hbm, z2_hbm, zeros_hbm, out_hbm, sidx, didx,
                  rows, acc, semg, sems):
        c = lax.axis_index("c")
        s = lax.axis_index("s")

        zbase = s * STRIPE
        pltpu.sync_copy(zeros_hbm, acc.at[pl.ds(zbase, STRIPE)])
        plsc.subcore_barrier()

        tilebase = s * RPT

        def chunk(i, carry):
            r0 = tilebase + i * CHUNK
            pltpu.sync_copy(src_hbm.at[c, pl.ds(r0, CHUNK)], sidx)
            pltpu.sync_copy(dst_hbm.at[pl.ds(r0, CHUNK)], didx)
            gathers = [pltpu.async_copy(z2_hbm.at[sidx.at[j]], rows.at[j],
                                        semg)
                       for j in range(CHUNK)]
            for gcp in gathers:
                gcp.wait()
            scats = [pltpu.async_copy(rows.at[j], acc.at[didx.at[j]], sems,
                                      add=True)
                     for j in range(CHUNK)]
            for scp in scats:
                scp.wait()
            return carry
        lax.fori_loop(0, NCHUNKS, chunk, 0)
        plsc.subcore_barrier()

        ob = s * STRIPE
        pltpu.sync_copy(acc.at[pl.ds(ob, STRIPE)],
                        out_hbm.at[c, pl.ds(ob, STRIPE)])
    return _seg_body


@functools.cache
def _segment_sum_sc(h):
    return pl.kernel(
        _make_seg_body(h),
        out_type=jax.ShapeDtypeStruct((2, ACC_ROWS, h), jnp.float32),
        mesh=plsc.VectorSubcoreMesh(core_axis_name="c",
                                    subcore_axis_name="s"),
        compiler_params=pltpu.CompilerParams(use_tc_tiling_on_sc=False),
        scratch_types=[
            pltpu.VMEM((CHUNK, 128), jnp.int32),
            pltpu.VMEM((CHUNK, 128), jnp.int32),
            pltpu.VMEM((CHUNK, 128, h), jnp.float32),
            pltpu.VMEM_SHARED((ACC_ROWS, h), jnp.float32),
            pltpu.SemaphoreType.DMA,
            pltpu.SemaphoreType.DMA,
        ],
    )


def _segment_sum(src_ab, dst_r, z_split):
    h = z_split.shape[2]
    zeros = jnp.zeros((STRIPE, h), jnp.float32)
    return _segment_sum_sc(h)(src_ab, dst_r, z_split.reshape(2 * N, h),
                              zeros)


def _stats_accum(st_ref, t):
    @pl.when(pl.program_id(0) == 0)
    def _():
        st_ref[...] = jnp.zeros_like(st_ref)
    s1 = jnp.sum(t, axis=0, keepdims=True)
    s2 = jnp.sum(t * t, axis=0, keepdims=True)
    st_ref[...] += jnp.concatenate(
        [s1, s2, jnp.zeros((6, EMB), jnp.float32)], axis=0)


def _mlp0_body(x_ref, agg_ref, w1_ref, b1_ref, w2_ref, b2_ref, t_ref,
               st_ref):
    aggf = jnp.concatenate([agg_ref[0], agg_ref[1]], axis=1)[:, :10]
    h = x_ref[...] + aggf
    h = jnp.maximum(jnp.dot(h, w1_ref[...],
                            preferred_element_type=jnp.float32)
                    + b1_ref[0:1, :], 0.0)
    t = jnp.dot(h, w2_ref[...], preferred_element_type=jnp.float32) \
        + b2_ref[0:1, :]
    t_ref[...] = t
    _stats_accum(st_ref, t)


def _mlp0(x, agg, w1, b1, w2, b2):
    return pl.pallas_call(
        _mlp0_body,
        grid=(N // RB,),
        in_specs=[pl.BlockSpec((RB, 10), lambda i: (i, 0)),
                  pl.BlockSpec((2, RB, 8), lambda i: (0, i, 0)),
                  pl.BlockSpec((10, EMB), lambda i: (0, 0)),
                  pl.BlockSpec((8, EMB), lambda i: (0, 0)),
                  pl.BlockSpec((EMB, EMB), lambda i: (0, 0)),
                  pl.BlockSpec((8, EMB), lambda i: (0, 0))],
        out_specs=[pl.BlockSpec((RB, EMB), lambda i: (i, 0)),
                   pl.BlockSpec((8, EMB), lambda i: (0, 0))],
        out_shape=[jax.ShapeDtypeStruct((N, EMB), jnp.float32),
                   jax.ShapeDtypeStruct((8, EMB), jnp.float32)],
    )(x, agg, w1, b1, w2, b2)


def _mlp_body(z_ref, agg_ref, w1_ref, b1_ref, w2_ref, b2_ref, t_ref,
              st_ref):
    zf = jnp.concatenate([z_ref[0], z_ref[1]], axis=1)
    aggf = jnp.concatenate([agg_ref[0], agg_ref[1]], axis=1)
    h = zf + aggf
    h = jnp.maximum(jnp.dot(h, w1_ref[...],
                            preferred_element_type=jnp.float32)
                    + b1_ref[0:1, :], 0.0)
    t = jnp.dot(h, w2_ref[...], preferred_element_type=jnp.float32) \
        + b2_ref[0:1, :]
    t_ref[...] = t
    _stats_accum(st_ref, t)


def _mlp(z_split, agg, w1, b1, w2, b2):
    return pl.pallas_call(
        _mlp_body,
        grid=(N // RB,),
        in_specs=[pl.BlockSpec((2, RB, HALF), lambda i: (0, i, 0)),
                  pl.BlockSpec((2, RB, HALF), lambda i: (0, i, 0)),
                  pl.BlockSpec((EMB, EMB), lambda i: (0, 0)),
                  pl.BlockSpec((8, EMB), lambda i: (0, 0)),
                  pl.BlockSpec((EMB, EMB), lambda i: (0, 0)),
                  pl.BlockSpec((8, EMB), lambda i: (0, 0))],
        out_specs=[pl.BlockSpec((RB, EMB), lambda i: (i, 0)),
                   pl.BlockSpec((8, EMB), lambda i: (0, 0))],
        out_shape=[jax.ShapeDtypeStruct((N, EMB), jnp.float32),
                   jax.ShapeDtypeStruct((8, EMB), jnp.float32)],
    )(z_split, agg, w1, b1, w2, b2)


def _norm_body(t_ref, sc_ref, sh_ref, o_ref):
    z = jnp.maximum(t_ref[...] * sc_ref[0:1, :] + sh_ref[0:1, :], 0.0)
    o_ref[0] = z[:, :HALF]
    o_ref[1] = z[:, HALF:]


def _norm(t, scale, shift):
    return pl.pallas_call(
        _norm_body,
        grid=(N // RB,),
        in_specs=[pl.BlockSpec((RB, EMB), lambda i: (i, 0)),
                  pl.BlockSpec((8, EMB), lambda i: (0, 0)),
                  pl.BlockSpec((8, EMB), lambda i: (0, 0))],
        out_specs=pl.BlockSpec((2, RB, HALF), lambda i: (0, i, 0)),
        out_shape=jax.ShapeDtypeStruct((2, N, HALF), jnp.float32),
    )(t, scale, shift)


def _norm_pool_body(t_ref, sc_ref, sh_ref, bat_ref, z_ref, gs_ref, gc_ref):
    z = jnp.maximum(t_ref[...] * sc_ref[0:1, :] + sh_ref[0:1, :], 0.0)
    z_ref[...] = z
    labels = bat_ref[0, 0, :]
    gids = lax.broadcasted_iota(jnp.int32, (RB, G), 1)
    oh = (labels[:, None] == gids).astype(jnp.float32)

    @pl.when(pl.program_id(0) == 0)
    def _():
        gs_ref[...] = jnp.zeros_like(gs_ref)
        gc_ref[...] = jnp.zeros_like(gc_ref)
    gs_ref[...] += lax.dot_general(oh, z, (((0,), (0,)), ((), ())),
                                   preferred_element_type=jnp.float32,
                                   precision=lax.Precision.HIGHEST)
    cnt = jnp.sum(oh, axis=0, keepdims=True)
    gc_ref[...] += jnp.concatenate(
        [cnt, jnp.zeros((7, G), jnp.float32)], axis=0)


def _norm_pool(t, scale, shift, batch3):
    return pl.pallas_call(
        _norm_pool_body,
        grid=(N // RB,),
        in_specs=[pl.BlockSpec((RB, EMB), lambda i: (i, 0)),
                  pl.BlockSpec((8, EMB), lambda i: (0, 0)),
                  pl.BlockSpec((8, EMB), lambda i: (0, 0)),
                  pl.BlockSpec((1, 1, RB), lambda i: (i, 0, 0))],
        out_specs=[pl.BlockSpec((RB, EMB), lambda i: (i, 0)),
                   pl.BlockSpec((G, EMB), lambda i: (0, 0)),
                   pl.BlockSpec((8, G), lambda i: (0, 0))],
        out_shape=[jax.ShapeDtypeStruct((N, EMB), jnp.float32),
                   jax.ShapeDtypeStruct((G, EMB), jnp.float32),
                   jax.ShapeDtypeStruct((8, G), jnp.float32)],
    )(t, scale, shift, batch3)


def _bn_coeffs(stats, gamma, beta):
    mean = stats[0] / N
    var = stats[1] / N - mean * mean
    rstd = lax.rsqrt(var + 1e-5)
    scale = gamma * rstd
    shift = beta - mean * scale
    return (jnp.broadcast_to(scale[None, :], (8, EMB)),
            jnp.broadcast_to(shift[None, :], (8, EMB)))


def kernel(x, edge_index, batch,
           W1_0, b1_0, W2_0, b2_0, gamma_0, beta_0,
           W1_1, b1_1, W2_1, b2_1, gamma_1, beta_1,
           W1_2, b1_2, W2_2, b2_2, gamma_2, beta_2):
    src = edge_index[0]
    dst = edge_index[1]
    pad = E2 - E
    zpad = jnp.zeros((pad,), jnp.int32)
    src_ab = jnp.stack([
        jnp.concatenate([src, zpad]),
        jnp.concatenate([src + N, zpad]),
    ]).reshape(2, ROWS, 128)
    dst_r = jnp.concatenate([dst, jnp.full((pad,), N, jnp.int32)]) \
        .reshape(ROWS, 128)
    batch3 = batch.reshape(N // RB, 1, RB)
    x16 = jnp.pad(x, ((0, 0), (0, 6)))
    x_split = jnp.stack([x16[:, :8], x16[:, 8:]])

    params = [(W1_0, b1_0, W2_0, b2_0, gamma_0, beta_0),
              (W1_1, b1_1, W2_1, b2_1, gamma_1, beta_1),
              (W1_2, b1_2, W2_2, b2_2, gamma_2, beta_2)]

    z_split = None
    node_rep = None
    for i, (w1, b1, w2, b2, gamma, beta) in enumerate(params):
        b18 = jnp.broadcast_to(b1[None, :], (8, EMB))
        b28 = jnp.broadcast_to(b2[None, :], (8, EMB))
        if i == 0:
            agg = _segment_sum(src_ab, dst_r, x_split)
            t, stats = _mlp0(x, agg, w1, b18, w2, b28)
        else:
            agg = _segment_sum(src_ab, dst_r, z_split)
            t, stats = _mlp(z_split, agg, w1, b18, w2, b28)
        scale8, shift8 = _bn_coeffs(stats, gamma, beta)
        if i < 2:
            z_split = _norm(t, scale8, shift8)
        else:
            node_rep, gsum, gcnt = _norm_pool(t, scale8, shift8, batch3)
    counts = gcnt[0]
    graph_rep = gsum / jnp.maximum(counts, 1.0)[:, None]
    return node_rep, graph_rep


# speedup vs baseline: 6.9504x; 1.0177x over previous
import functools

import jax
import jax.numpy as jnp
from jax import lax
from jax.experimental import pallas as pl
from jax.experimental.pallas import tpu as pltpu
from jax.experimental.pallas import tpu_sc as plsc

N = 100000
E = 1600000
EMB = 32
G = 64
HALF = 16
NSUB = 16
RPT = 800
ROWS = NSUB * RPT
E2 = ROWS * 128
CHUNK = 5
NPAIRS = RPT // (2 * CHUNK)
ACC_ROWS = 100096
STRIPE = ACC_ROWS // NSUB
NZCOPY = 16
ZROWS = STRIPE // NZCOPY
RB = 1000


def _make_seg_body(h):
    def _seg_body(src_hbm, dst_hbm, z2_hbm, zeros_hbm, out_hbm, sidx, didx,
                  rows, acc, semg, sems):
        c = lax.axis_index("c")
        s = lax.axis_index("s")

        zbase = s * STRIPE
        pltpu.sync_copy(zeros_hbm, acc.at[pl.ds(zbase, STRIPE)])
        plsc.subcore_barrier()

        tilebase = s * RPT

        def load_idx(r0, b):
            pltpu.sync_copy(src_hbm.at[c, pl.ds(r0, CHUNK)], sidx.at[b])
            pltpu.sync_copy(dst_hbm.at[pl.ds(r0, CHUNK)], didx.at[b])

        def fire_gathers(b):
            return [pltpu.async_copy(z2_hbm.at[sidx.at[b, j]],
                                     rows.at[b, j], semg)
                    for j in range(CHUNK)]

        def fire_scatters(b):
            return [pltpu.async_copy(rows.at[b, j], acc.at[didx.at[b, j]],
                                     sems, add=True)
                    for j in range(CHUNK)]

        def pair(i, carry):
            r0 = tilebase + i * 2 * CHUNK
            load_idx(r0, 0)
            ga = fire_gathers(0)
            load_idx(r0 + CHUNK, 1)
            for gcp in ga:
                gcp.wait()
            sa = fire_scatters(0)
            gb = fire_gathers(1)
            for gcp in gb:
                gcp.wait()
            sb = fire_scatters(1)
            for scp in sa:
                scp.wait()
            for scp in sb:
                scp.wait()
            return carry
        lax.fori_loop(0, NPAIRS, pair, 0)
        plsc.subcore_barrier()

        ob = s * STRIPE
        pltpu.sync_copy(acc.at[pl.ds(ob, STRIPE)],
                        out_hbm.at[c, pl.ds(ob, STRIPE)])
    return _seg_body


@functools.cache
def _segment_sum_sc(h):
    return pl.kernel(
        _make_seg_body(h),
        out_type=jax.ShapeDtypeStruct((2, ACC_ROWS, h), jnp.float32),
        mesh=plsc.VectorSubcoreMesh(core_axis_name="c",
                                    subcore_axis_name="s"),
        compiler_params=pltpu.CompilerParams(use_tc_tiling_on_sc=False),
        scratch_types=[
            pltpu.VMEM((2, CHUNK, 128), jnp.int32),
            pltpu.VMEM((2, CHUNK, 128), jnp.int32),
            pltpu.VMEM((2, CHUNK, 128, h), jnp.float32),
            pltpu.VMEM_SHARED((ACC_ROWS, h), jnp.float32),
            pltpu.SemaphoreType.DMA,
            pltpu.SemaphoreType.DMA,
        ],
    )


def _segment_sum(src_ab, dst_r, z_split):
    h = z_split.shape[2]
    zeros = jnp.zeros((STRIPE, h), jnp.float32)
    return _segment_sum_sc(h)(src_ab, dst_r, z_split.reshape(2 * N, h),
                              zeros)


def _stats_accum(st_ref, t):
    @pl.when(pl.program_id(0) == 0)
    def _():
        st_ref[...] = jnp.zeros_like(st_ref)
    s1 = jnp.sum(t, axis=0, keepdims=True)
    s2 = jnp.sum(t * t, axis=0, keepdims=True)
    st_ref[...] += jnp.concatenate(
        [s1, s2, jnp.zeros((6, EMB), jnp.float32)], axis=0)


def _mlp0_body(x_ref, agg_ref, w1_ref, b1_ref, w2_ref, b2_ref, t_ref,
               st_ref):
    aggf = jnp.concatenate([agg_ref[0], agg_ref[1]], axis=1)[:, :10]
    h = x_ref[...] + aggf
    h = jnp.maximum(jnp.dot(h, w1_ref[...],
                            preferred_element_type=jnp.float32)
                    + b1_ref[0:1, :], 0.0)
    t = jnp.dot(h, w2_ref[...], preferred_element_type=jnp.float32) \
        + b2_ref[0:1, :]
    t_ref[...] = t
    _stats_accum(st_ref, t)


def _mlp0(x, agg, w1, b1, w2, b2):
    return pl.pallas_call(
        _mlp0_body,
        grid=(N // RB,),
        in_specs=[pl.BlockSpec((RB, 10), lambda i: (i, 0)),
                  pl.BlockSpec((2, RB, 8), lambda i: (0, i, 0)),
                  pl.BlockSpec((10, EMB), lambda i: (0, 0)),
                  pl.BlockSpec((8, EMB), lambda i: (0, 0)),
                  pl.BlockSpec((EMB, EMB), lambda i: (0, 0)),
                  pl.BlockSpec((8, EMB), lambda i: (0, 0))],
        out_specs=[pl.BlockSpec((RB, EMB), lambda i: (i, 0)),
                   pl.BlockSpec((8, EMB), lambda i: (0, 0))],
        out_shape=[jax.ShapeDtypeStruct((N, EMB), jnp.float32),
                   jax.ShapeDtypeStruct((8, EMB), jnp.float32)],
    )(x, agg, w1, b1, w2, b2)


def _mlp_body(z_ref, agg_ref, w1_ref, b1_ref, w2_ref, b2_ref, t_ref,
              st_ref):
    zf = jnp.concatenate([z_ref[0], z_ref[1]], axis=1)
    aggf = jnp.concatenate([agg_ref[0], agg_ref[1]], axis=1)
    h = zf + aggf
    h = jnp.maximum(jnp.dot(h, w1_ref[...],
                            preferred_element_type=jnp.float32)
                    + b1_ref[0:1, :], 0.0)
    t = jnp.dot(h, w2_ref[...], preferred_element_type=jnp.float32) \
        + b2_ref[0:1, :]
    t_ref[...] = t
    _stats_accum(st_ref, t)


def _mlp(z_split, agg, w1, b1, w2, b2):
    return pl.pallas_call(
        _mlp_body,
        grid=(N // RB,),
        in_specs=[pl.BlockSpec((2, RB, HALF), lambda i: (0, i, 0)),
                  pl.BlockSpec((2, RB, HALF), lambda i: (0, i, 0)),
                  pl.BlockSpec((EMB, EMB), lambda i: (0, 0)),
                  pl.BlockSpec((8, EMB), lambda i: (0, 0)),
                  pl.BlockSpec((EMB, EMB), lambda i: (0, 0)),
                  pl.BlockSpec((8, EMB), lambda i: (0, 0))],
        out_specs=[pl.BlockSpec((RB, EMB), lambda i: (i, 0)),
                   pl.BlockSpec((8, EMB), lambda i: (0, 0))],
        out_shape=[jax.ShapeDtypeStruct((N, EMB), jnp.float32),
                   jax.ShapeDtypeStruct((8, EMB), jnp.float32)],
    )(z_split, agg, w1, b1, w2, b2)


def _norm_body(t_ref, sc_ref, sh_ref, o_ref):
    z = jnp.maximum(t_ref[...] * sc_ref[0:1, :] + sh_ref[0:1, :], 0.0)
    o_ref[0] = z[:, :HALF]
    o_ref[1] = z[:, HALF:]


def _norm(t, scale, shift):
    return pl.pallas_call(
        _norm_body,
        grid=(N // RB,),
        in_specs=[pl.BlockSpec((RB, EMB), lambda i: (i, 0)),
                  pl.BlockSpec((8, EMB), lambda i: (0, 0)),
                  pl.BlockSpec((8, EMB), lambda i: (0, 0))],
        out_specs=pl.BlockSpec((2, RB, HALF), lambda i: (0, i, 0)),
        out_shape=jax.ShapeDtypeStruct((2, N, HALF), jnp.float32),
    )(t, scale, shift)


def _norm_pool_body(t_ref, sc_ref, sh_ref, bat_ref, z_ref, gs_ref, gc_ref):
    z = jnp.maximum(t_ref[...] * sc_ref[0:1, :] + sh_ref[0:1, :], 0.0)
    z_ref[...] = z
    labels = bat_ref[0, 0, :]
    gids = lax.broadcasted_iota(jnp.int32, (RB, G), 1)
    oh = (labels[:, None] == gids).astype(jnp.float32)

    @pl.when(pl.program_id(0) == 0)
    def _():
        gs_ref[...] = jnp.zeros_like(gs_ref)
        gc_ref[...] = jnp.zeros_like(gc_ref)
    gs_ref[...] += lax.dot_general(oh, z, (((0,), (0,)), ((), ())),
                                   preferred_element_type=jnp.float32,
                                   precision=lax.Precision.HIGHEST)
    cnt = jnp.sum(oh, axis=0, keepdims=True)
    gc_ref[...] += jnp.concatenate(
        [cnt, jnp.zeros((7, G), jnp.float32)], axis=0)


def _norm_pool(t, scale, shift, batch3):
    return pl.pallas_call(
        _norm_pool_body,
        grid=(N // RB,),
        in_specs=[pl.BlockSpec((RB, EMB), lambda i: (i, 0)),
                  pl.BlockSpec((8, EMB), lambda i: (0, 0)),
                  pl.BlockSpec((8, EMB), lambda i: (0, 0)),
                  pl.BlockSpec((1, 1, RB), lambda i: (i, 0, 0))],
        out_specs=[pl.BlockSpec((RB, EMB), lambda i: (i, 0)),
                   pl.BlockSpec((G, EMB), lambda i: (0, 0)),
                   pl.BlockSpec((8, G), lambda i: (0, 0))],
        out_shape=[jax.ShapeDtypeStruct((N, EMB), jnp.float32),
                   jax.ShapeDtypeStruct((G, EMB), jnp.float32),
                   jax.ShapeDtypeStruct((8, G), jnp.float32)],
    )(t, scale, shift, batch3)


def _bn_coeffs(stats, gamma, beta):
    mean = stats[0] / N
    var = stats[1] / N - mean * mean
    rstd = lax.rsqrt(var + 1e-5)
    scale = gamma * rstd
    shift = beta - mean * scale
    return (jnp.broadcast_to(scale[None, :], (8, EMB)),
            jnp.broadcast_to(shift[None, :], (8, EMB)))


def kernel(x, edge_index, batch,
           W1_0, b1_0, W2_0, b2_0, gamma_0, beta_0,
           W1_1, b1_1, W2_1, b2_1, gamma_1, beta_1,
           W1_2, b1_2, W2_2, b2_2, gamma_2, beta_2):
    src = edge_index[0]
    dst = edge_index[1]
    pad = E2 - E
    zpad = jnp.zeros((pad,), jnp.int32)
    src_ab = jnp.stack([
        jnp.concatenate([src, zpad]),
        jnp.concatenate([src + N, zpad]),
    ]).reshape(2, ROWS, 128)
    dst_r = jnp.concatenate([dst, jnp.full((pad,), N, jnp.int32)]) \
        .reshape(ROWS, 128)
    batch3 = batch.reshape(N // RB, 1, RB)
    x16 = jnp.pad(x, ((0, 0), (0, 6)))
    x_split = jnp.stack([x16[:, :8], x16[:, 8:]])

    params = [(W1_0, b1_0, W2_0, b2_0, gamma_0, beta_0),
              (W1_1, b1_1, W2_1, b2_1, gamma_1, beta_1),
              (W1_2, b1_2, W2_2, b2_2, gamma_2, beta_2)]

    z_split = None
    node_rep = None
    for i, (w1, b1, w2, b2, gamma, beta) in enumerate(params):
        b18 = jnp.broadcast_to(b1[None, :], (8, EMB))
        b28 = jnp.broadcast_to(b2[None, :], (8, EMB))
        if i == 0:
            agg = _segment_sum(src_ab, dst_r, x_split)
            t, stats = _mlp0(x, agg, w1, b18, w2, b28)
        else:
            agg = _segment_sum(src_ab, dst_r, z_split)
            t, stats = _mlp(z_split, agg, w1, b18, w2, b28)
        scale8, shift8 = _bn_coeffs(stats, gamma, beta)
        if i < 2:
            z_split = _norm(t, scale8, shift8)
        else:
            node_rep, gsum, gcnt = _norm_pool(t, scale8, shift8, batch3)
    counts = gcnt[0]
    graph_rep = gsum / jnp.maximum(counts, 1.0)[:, None]
    return node_rep, graph_rep


# speedup vs baseline: 7.0423x; 1.0132x over previous
import functools

import jax
import jax.numpy as jnp
from jax import lax
from jax.experimental import pallas as pl
from jax.experimental.pallas import tpu as pltpu
from jax.experimental.pallas import tpu_sc as plsc

N = 100000
E = 1600000
EMB = 32
G = 64
HALF = 16
NSUB = 16
RPT = 800
ROWS = NSUB * RPT
E2 = ROWS * 128
CHUNK = 5
NPAIRS = RPT // (2 * CHUNK)
ACC_ROWS = 100096
STRIPE = ACC_ROWS // NSUB
NZCOPY = 16
ZROWS = STRIPE // NZCOPY
RB = 1000


def _make_seg_body(h):
    def _seg_body(src_hbm, dst_hbm, z2_hbm, zeros_hbm, out_hbm, sidx, didx,
                  rows, acc, semg, sems):
        c = lax.axis_index("c")
        s = lax.axis_index("s")

        zbase = s * STRIPE
        pltpu.sync_copy(zeros_hbm, acc.at[pl.ds(zbase, STRIPE)])
        plsc.subcore_barrier()

        tilebase = s * RPT * 128
        ECH = CHUNK * 128

        def load_idx(e0, b):
            pltpu.sync_copy(src_hbm.at[c, pl.ds(e0, ECH)], sidx.at[b])
            pltpu.sync_copy(dst_hbm.at[pl.ds(e0, ECH)], didx.at[b])

        def fire_gathers(b):
            return [pltpu.async_copy(z2_hbm.at[sidx.at[b]], rows.at[b],
                                     semg)]

        def fire_scatters(b):
            return [pltpu.async_copy(rows.at[b], acc.at[didx.at[b]], sems,
                                     add=True)]

        def pair(i, carry):
            e0 = tilebase + i * 2 * ECH
            load_idx(e0, 0)
            ga = fire_gathers(0)
            load_idx(e0 + ECH, 1)
            for gcp in ga:
                gcp.wait()
            sa = fire_scatters(0)
            gb = fire_gathers(1)
            for gcp in gb:
                gcp.wait()
            sb = fire_scatters(1)
            for scp in sa:
                scp.wait()
            for scp in sb:
                scp.wait()
            return carry
        lax.fori_loop(0, NPAIRS, pair, 0)
        plsc.subcore_barrier()

        ob = s * STRIPE
        pltpu.sync_copy(acc.at[pl.ds(ob, STRIPE)],
                        out_hbm.at[c, pl.ds(ob, STRIPE)])
    return _seg_body


@functools.cache
def _segment_sum_sc(h):
    return pl.kernel(
        _make_seg_body(h),
        out_type=jax.ShapeDtypeStruct((2, ACC_ROWS, h), jnp.float32),
        mesh=plsc.VectorSubcoreMesh(core_axis_name="c",
                                    subcore_axis_name="s"),
        compiler_params=pltpu.CompilerParams(use_tc_tiling_on_sc=False),
        scratch_types=[
            pltpu.VMEM((2, CHUNK * 128), jnp.int32),
            pltpu.VMEM((2, CHUNK * 128), jnp.int32),
            pltpu.VMEM((2, CHUNK * 128, h), jnp.float32),
            pltpu.VMEM_SHARED((ACC_ROWS, h), jnp.float32),
            pltpu.SemaphoreType.DMA,
            pltpu.SemaphoreType.DMA,
        ],
    )


def _segment_sum(src_ab, dst_r, z_split):
    h = z_split.shape[2]
    zeros = jnp.zeros((STRIPE, h), jnp.float32)
    return _segment_sum_sc(h)(src_ab, dst_r, z_split.reshape(2 * N, h),
                              zeros)


def _stats_accum(st_ref, t):
    @pl.when(pl.program_id(0) == 0)
    def _():
        st_ref[...] = jnp.zeros_like(st_ref)
    s1 = jnp.sum(t, axis=0, keepdims=True)
    s2 = jnp.sum(t * t, axis=0, keepdims=True)
    st_ref[...] += jnp.concatenate(
        [s1, s2, jnp.zeros((6, EMB), jnp.float32)], axis=0)


def _mlp0_body(x_ref, agg_ref, w1_ref, b1_ref, w2_ref, b2_ref, t_ref,
               st_ref):
    aggf = jnp.concatenate([agg_ref[0], agg_ref[1]], axis=1)[:, :10]
    h = x_ref[...] + aggf
    h = jnp.maximum(jnp.dot(h, w1_ref[...],
                            preferred_element_type=jnp.float32)
                    + b1_ref[0:1, :], 0.0)
    t = jnp.dot(h, w2_ref[...], preferred_element_type=jnp.float32) \
        + b2_ref[0:1, :]
    t_ref[...] = t
    _stats_accum(st_ref, t)


def _mlp0(x, agg, w1, b1, w2, b2):
    return pl.pallas_call(
        _mlp0_body,
        grid=(N // RB,),
        in_specs=[pl.BlockSpec((RB, 10), lambda i: (i, 0)),
                  pl.BlockSpec((2, RB, 8), lambda i: (0, i, 0)),
                  pl.BlockSpec((10, EMB), lambda i: (0, 0)),
                  pl.BlockSpec((8, EMB), lambda i: (0, 0)),
                  pl.BlockSpec((EMB, EMB), lambda i: (0, 0)),
                  pl.BlockSpec((8, EMB), lambda i: (0, 0))],
        out_specs=[pl.BlockSpec((RB, EMB), lambda i: (i, 0)),
                   pl.BlockSpec((8, EMB), lambda i: (0, 0))],
        out_shape=[jax.ShapeDtypeStruct((N, EMB), jnp.float32),
                   jax.ShapeDtypeStruct((8, EMB), jnp.float32)],
    )(x, agg, w1, b1, w2, b2)


def _mlp_body(z_ref, agg_ref, w1_ref, b1_ref, w2_ref, b2_ref, t_ref,
              st_ref):
    zf = jnp.concatenate([z_ref[0], z_ref[1]], axis=1)
    aggf = jnp.concatenate([agg_ref[0], agg_ref[1]], axis=1)
    h = zf + aggf
    h = jnp.maximum(jnp.dot(h, w1_ref[...],
                            preferred_element_type=jnp.float32)
                    + b1_ref[0:1, :], 0.0)
    t = jnp.dot(h, w2_ref[...], preferred_element_type=jnp.float32) \
        + b2_ref[0:1, :]
    t_ref[...] = t
    _stats_accum(st_ref, t)


def _mlp(z_split, agg, w1, b1, w2, b2):
    return pl.pallas_call(
        _mlp_body,
        grid=(N // RB,),
        in_specs=[pl.BlockSpec((2, RB, HALF), lambda i: (0, i, 0)),
                  pl.BlockSpec((2, RB, HALF), lambda i: (0, i, 0)),
                  pl.BlockSpec((EMB, EMB), lambda i: (0, 0)),
                  pl.BlockSpec((8, EMB), lambda i: (0, 0)),
                  pl.BlockSpec((EMB, EMB), lambda i: (0, 0)),
                  pl.BlockSpec((8, EMB), lambda i: (0, 0))],
        out_specs=[pl.BlockSpec((RB, EMB), lambda i: (i, 0)),
                   pl.BlockSpec((8, EMB), lambda i: (0, 0))],
        out_shape=[jax.ShapeDtypeStruct((N, EMB), jnp.float32),
                   jax.ShapeDtypeStruct((8, EMB), jnp.float32)],
    )(z_split, agg, w1, b1, w2, b2)


def _norm_body(t_ref, sc_ref, sh_ref, o_ref):
    z = jnp.maximum(t_ref[...] * sc_ref[0:1, :] + sh_ref[0:1, :], 0.0)
    o_ref[0] = z[:, :HALF]
    o_ref[1] = z[:, HALF:]


def _norm(t, scale, shift):
    return pl.pallas_call(
        _norm_body,
        grid=(N // RB,),
        in_specs=[pl.BlockSpec((RB, EMB), lambda i: (i, 0)),
                  pl.BlockSpec((8, EMB), lambda i: (0, 0)),
                  pl.BlockSpec((8, EMB), lambda i: (0, 0))],
        out_specs=pl.BlockSpec((2, RB, HALF), lambda i: (0, i, 0)),
        out_shape=jax.ShapeDtypeStruct((2, N, HALF), jnp.float32),
    )(t, scale, shift)


def _norm_pool_body(t_ref, sc_ref, sh_ref, bat_ref, z_ref, gs_ref, gc_ref):
    z = jnp.maximum(t_ref[...] * sc_ref[0:1, :] + sh_ref[0:1, :], 0.0)
    z_ref[...] = z
    labels = bat_ref[0, 0, :]
    gids = lax.broadcasted_iota(jnp.int32, (RB, G), 1)
    oh = (labels[:, None] == gids).astype(jnp.float32)

    @pl.when(pl.program_id(0) == 0)
    def _():
        gs_ref[...] = jnp.zeros_like(gs_ref)
        gc_ref[...] = jnp.zeros_like(gc_ref)
    gs_ref[...] += lax.dot_general(oh, z, (((0,), (0,)), ((), ())),
                                   preferred_element_type=jnp.float32,
                                   precision=lax.Precision.HIGHEST)
    cnt = jnp.sum(oh, axis=0, keepdims=True)
    gc_ref[...] += jnp.concatenate(
        [cnt, jnp.zeros((7, G), jnp.float32)], axis=0)


def _norm_pool(t, scale, shift, batch3):
    return pl.pallas_call(
        _norm_pool_body,
        grid=(N // RB,),
        in_specs=[pl.BlockSpec((RB, EMB), lambda i: (i, 0)),
                  pl.BlockSpec((8, EMB), lambda i: (0, 0)),
                  pl.BlockSpec((8, EMB), lambda i: (0, 0)),
                  pl.BlockSpec((1, 1, RB), lambda i: (i, 0, 0))],
        out_specs=[pl.BlockSpec((RB, EMB), lambda i: (i, 0)),
                   pl.BlockSpec((G, EMB), lambda i: (0, 0)),
                   pl.BlockSpec((8, G), lambda i: (0, 0))],
        out_shape=[jax.ShapeDtypeStruct((N, EMB), jnp.float32),
                   jax.ShapeDtypeStruct((G, EMB), jnp.float32),
                   jax.ShapeDtypeStruct((8, G), jnp.float32)],
    )(t, scale, shift, batch3)


def _bn_coeffs(stats, gamma, beta):
    mean = stats[0] / N
    var = stats[1] / N - mean * mean
    rstd = lax.rsqrt(var + 1e-5)
    scale = gamma * rstd
    shift = beta - mean * scale
    return (jnp.broadcast_to(scale[None, :], (8, EMB)),
            jnp.broadcast_to(shift[None, :], (8, EMB)))


def kernel(x, edge_index, batch,
           W1_0, b1_0, W2_0, b2_0, gamma_0, beta_0,
           W1_1, b1_1, W2_1, b2_1, gamma_1, beta_1,
           W1_2, b1_2, W2_2, b2_2, gamma_2, beta_2):
    src = edge_index[0]
    dst = edge_index[1]
    pad = E2 - E
    zpad = jnp.zeros((pad,), jnp.int32)
    src_ab = jnp.stack([
        jnp.concatenate([src, zpad]),
        jnp.concatenate([src + N, zpad]),
    ])
    dst_r = jnp.concatenate([dst, jnp.full((pad,), N, jnp.int32)])
    batch3 = batch.reshape(N // RB, 1, RB)
    x16 = jnp.pad(x, ((0, 0), (0, 6)))
    x_split = jnp.stack([x16[:, :8], x16[:, 8:]])

    params = [(W1_0, b1_0, W2_0, b2_0, gamma_0, beta_0),
              (W1_1, b1_1, W2_1, b2_1, gamma_1, beta_1),
              (W1_2, b1_2, W2_2, b2_2, gamma_2, beta_2)]

    z_split = None
    node_rep = None
    for i, (w1, b1, w2, b2, gamma, beta) in enumerate(params):
        b18 = jnp.broadcast_to(b1[None, :], (8, EMB))
        b28 = jnp.broadcast_to(b2[None, :], (8, EMB))
        if i == 0:
            agg = _segment_sum(src_ab, dst_r, x_split)
            t, stats = _mlp0(x, agg, w1, b18, w2, b28)
        else:
            agg = _segment_sum(src_ab, dst_r, z_split)
            t, stats = _mlp(z_split, agg, w1, b18, w2, b28)
        scale8, shift8 = _bn_coeffs(stats, gamma, beta)
        if i < 2:
            z_split = _norm(t, scale8, shift8)
        else:
            node_rep, gsum, gcnt = _norm_pool(t, scale8, shift8, batch3)
    counts = gcnt[0]
    graph_rep = gsum / jnp.maximum(counts, 1.0)[:, None]
    return node_rep, graph_rep


# speedup vs baseline: 7.7077x; 1.0945x over previous
import functools

import jax
import jax.numpy as jnp
from jax import lax
from jax.experimental import pallas as pl
from jax.experimental.pallas import tpu as pltpu
from jax.experimental.pallas import tpu_sc as plsc

N = 100000
E = 1600000
EMB = 32
G = 64
HALF = 16
NSUB = 16
RPT = 800
ROWS = NSUB * RPT
E2 = ROWS * 128
CHUNK = 5
NPAIRS = RPT // (2 * CHUNK)
ACC_ROWS = 100096
STRIPE = ACC_ROWS // NSUB
NZCOPY = 16
ZROWS = STRIPE // NZCOPY
RB = 10000


def _make_seg_body(h):
    def _seg_body(src_hbm, dst_hbm, z2_hbm, zeros_hbm, out_hbm, sidx, didx,
                  rows, acc, semg, sems):
        c = lax.axis_index("c")
        s = lax.axis_index("s")

        zbase = s * STRIPE
        pltpu.sync_copy(zeros_hbm, acc.at[pl.ds(zbase, STRIPE)])
        plsc.subcore_barrier()

        tilebase = s * RPT * 128
        ECH = CHUNK * 128

        def load_idx(e0, b):
            pltpu.sync_copy(src_hbm.at[c, pl.ds(e0, ECH)], sidx.at[b])
            pltpu.sync_copy(dst_hbm.at[pl.ds(e0, ECH)], didx.at[b])

        def fire_gathers(b):
            return [pltpu.async_copy(z2_hbm.at[sidx.at[b]], rows.at[b],
                                     semg)]

        def fire_scatters(b):
            return [pltpu.async_copy(rows.at[b], acc.at[didx.at[b]], sems,
                                     add=True)]

        def pair(i, carry):
            e0 = tilebase + i * 2 * ECH
            load_idx(e0, 0)
            ga = fire_gathers(0)
            load_idx(e0 + ECH, 1)
            for gcp in ga:
                gcp.wait()
            sa = fire_scatters(0)
            gb = fire_gathers(1)
            for gcp in gb:
                gcp.wait()
            sb = fire_scatters(1)
            for scp in sa:
                scp.wait()
            for scp in sb:
                scp.wait()
            return carry
        lax.fori_loop(0, NPAIRS, pair, 0)
        plsc.subcore_barrier()

        ob = s * STRIPE
        pltpu.sync_copy(acc.at[pl.ds(ob, STRIPE)],
                        out_hbm.at[c, pl.ds(ob, STRIPE)])
    return _seg_body


@functools.cache
def _segment_sum_sc(h):
    return pl.kernel(
        _make_seg_body(h),
        out_type=jax.ShapeDtypeStruct((2, ACC_ROWS, h), jnp.float32),
        mesh=plsc.VectorSubcoreMesh(core_axis_name="c",
                                    subcore_axis_name="s"),
        compiler_params=pltpu.CompilerParams(use_tc_tiling_on_sc=False),
        scratch_types=[
            pltpu.VMEM((2, CHUNK * 128), jnp.int32),
            pltpu.VMEM((2, CHUNK * 128), jnp.int32),
            pltpu.VMEM((2, CHUNK * 128, h), jnp.float32),
            pltpu.VMEM_SHARED((ACC_ROWS, h), jnp.float32),
            pltpu.SemaphoreType.DMA,
            pltpu.SemaphoreType.DMA,
        ],
    )


def _segment_sum(src_ab, dst_r, z_split):
    h = z_split.shape[2]
    zeros = jnp.zeros((STRIPE, h), jnp.float32)
    return _segment_sum_sc(h)(src_ab, dst_r, z_split.reshape(2 * N, h),
                              zeros)


def _stats_accum(st_ref, t):
    @pl.when(pl.program_id(0) == 0)
    def _():
        st_ref[...] = jnp.zeros_like(st_ref)
    s1 = jnp.sum(t, axis=0, keepdims=True)
    s2 = jnp.sum(t * t, axis=0, keepdims=True)
    st_ref[...] += jnp.concatenate(
        [s1, s2, jnp.zeros((6, EMB), jnp.float32)], axis=0)


def _mlp0_body(x_ref, agg_ref, w1_ref, b1_ref, w2_ref, b2_ref, t_ref,
               st_ref):
    aggf = jnp.concatenate([agg_ref[0], agg_ref[1]], axis=1)[:, :10]
    h = x_ref[...] + aggf
    h = jnp.maximum(jnp.dot(h, w1_ref[...],
                            preferred_element_type=jnp.float32)
                    + b1_ref[0:1, :], 0.0)
    t = jnp.dot(h, w2_ref[...], preferred_element_type=jnp.float32) \
        + b2_ref[0:1, :]
    t_ref[...] = t
    _stats_accum(st_ref, t)


def _mlp0(x, agg, w1, b1, w2, b2):
    return pl.pallas_call(
        _mlp0_body,
        grid=(N // RB,),
        in_specs=[pl.BlockSpec((RB, 10), lambda i: (i, 0)),
                  pl.BlockSpec((2, RB, 8), lambda i: (0, i, 0)),
                  pl.BlockSpec((10, EMB), lambda i: (0, 0)),
                  pl.BlockSpec((8, EMB), lambda i: (0, 0)),
                  pl.BlockSpec((EMB, EMB), lambda i: (0, 0)),
                  pl.BlockSpec((8, EMB), lambda i: (0, 0))],
        out_specs=[pl.BlockSpec((RB, EMB), lambda i: (i, 0)),
                   pl.BlockSpec((8, EMB), lambda i: (0, 0))],
        out_shape=[jax.ShapeDtypeStruct((N, EMB), jnp.float32),
                   jax.ShapeDtypeStruct((8, EMB), jnp.float32)],
    )(x, agg, w1, b1, w2, b2)


def _mlp_body(z_ref, agg_ref, w1_ref, b1_ref, w2_ref, b2_ref, t_ref,
              st_ref):
    zf = jnp.concatenate([z_ref[0], z_ref[1]], axis=1)
    aggf = jnp.concatenate([agg_ref[0], agg_ref[1]], axis=1)
    h = zf + aggf
    h = jnp.maximum(jnp.dot(h, w1_ref[...],
                            preferred_element_type=jnp.float32)
                    + b1_ref[0:1, :], 0.0)
    t = jnp.dot(h, w2_ref[...], preferred_element_type=jnp.float32) \
        + b2_ref[0:1, :]
    t_ref[...] = t
    _stats_accum(st_ref, t)


def _mlp(z_split, agg, w1, b1, w2, b2):
    return pl.pallas_call(
        _mlp_body,
        grid=(N // RB,),
        in_specs=[pl.BlockSpec((2, RB, HALF), lambda i: (0, i, 0)),
                  pl.BlockSpec((2, RB, HALF), lambda i: (0, i, 0)),
                  pl.BlockSpec((EMB, EMB), lambda i: (0, 0)),
                  pl.BlockSpec((8, EMB), lambda i: (0, 0)),
                  pl.BlockSpec((EMB, EMB), lambda i: (0, 0)),
                  pl.BlockSpec((8, EMB), lambda i: (0, 0))],
        out_specs=[pl.BlockSpec((RB, EMB), lambda i: (i, 0)),
                   pl.BlockSpec((8, EMB), lambda i: (0, 0))],
        out_shape=[jax.ShapeDtypeStruct((N, EMB), jnp.float32),
                   jax.ShapeDtypeStruct((8, EMB), jnp.float32)],
    )(z_split, agg, w1, b1, w2, b2)


def _norm_body(t_ref, sc_ref, sh_ref, o_ref):
    z = jnp.maximum(t_ref[...] * sc_ref[0:1, :] + sh_ref[0:1, :], 0.0)
    o_ref[0] = z[:, :HALF]
    o_ref[1] = z[:, HALF:]


def _norm(t, scale, shift):
    return pl.pallas_call(
        _norm_body,
        grid=(N // RB,),
        in_specs=[pl.BlockSpec((RB, EMB), lambda i: (i, 0)),
                  pl.BlockSpec((8, EMB), lambda i: (0, 0)),
                  pl.BlockSpec((8, EMB), lambda i: (0, 0))],
        out_specs=pl.BlockSpec((2, RB, HALF), lambda i: (0, i, 0)),
        out_shape=jax.ShapeDtypeStruct((2, N, HALF), jnp.float32),
    )(t, scale, shift)


def _norm_pool_body(t_ref, sc_ref, sh_ref, bat_ref, z_ref, gs_ref, gc_ref):
    z = jnp.maximum(t_ref[...] * sc_ref[0:1, :] + sh_ref[0:1, :], 0.0)
    z_ref[...] = z
    labels = bat_ref[0, 0, :]
    gids = lax.broadcasted_iota(jnp.int32, (RB, G), 1)
    oh = (labels[:, None] == gids).astype(jnp.float32)

    @pl.when(pl.program_id(0) == 0)
    def _():
        gs_ref[...] = jnp.zeros_like(gs_ref)
        gc_ref[...] = jnp.zeros_like(gc_ref)
    gs_ref[...] += lax.dot_general(oh, z, (((0,), (0,)), ((), ())),
                                   preferred_element_type=jnp.float32,
                                   precision=lax.Precision.HIGHEST)
    cnt = jnp.sum(oh, axis=0, keepdims=True)
    gc_ref[...] += jnp.concatenate(
        [cnt, jnp.zeros((7, G), jnp.float32)], axis=0)


def _norm_pool(t, scale, shift, batch3):
    return pl.pallas_call(
        _norm_pool_body,
        grid=(N // RB,),
        in_specs=[pl.BlockSpec((RB, EMB), lambda i: (i, 0)),
                  pl.BlockSpec((8, EMB), lambda i: (0, 0)),
                  pl.BlockSpec((8, EMB), lambda i: (0, 0)),
                  pl.BlockSpec((1, 1, RB), lambda i: (i, 0, 0))],
        out_specs=[pl.BlockSpec((RB, EMB), lambda i: (i, 0)),
                   pl.BlockSpec((G, EMB), lambda i: (0, 0)),
                   pl.BlockSpec((8, G), lambda i: (0, 0))],
        out_shape=[jax.ShapeDtypeStruct((N, EMB), jnp.float32),
                   jax.ShapeDtypeStruct((G, EMB), jnp.float32),
                   jax.ShapeDtypeStruct((8, G), jnp.float32)],
    )(t, scale, shift, batch3)


def _bn_coeffs(stats, gamma, beta):
    mean = stats[0] / N
    var = stats[1] / N - mean * mean
    rstd = lax.rsqrt(var + 1e-5)
    scale = gamma * rstd
    shift = beta - mean * scale
    return (jnp.broadcast_to(scale[None, :], (8, EMB)),
            jnp.broadcast_to(shift[None, :], (8, EMB)))


def kernel(x, edge_index, batch,
           W1_0, b1_0, W2_0, b2_0, gamma_0, beta_0,
           W1_1, b1_1, W2_1, b2_1, gamma_1, beta_1,
           W1_2, b1_2, W2_2, b2_2, gamma_2, beta_2):
    src = edge_index[0]
    dst = edge_index[1]
    pad = E2 - E
    zpad = jnp.zeros((pad,), jnp.int32)
    src_ab = jnp.stack([
        jnp.concatenate([src, zpad]),
        jnp.concatenate([src + N, zpad]),
    ])
    dst_r = jnp.concatenate([dst, jnp.full((pad,), N, jnp.int32)])
    batch3 = batch.reshape(N // RB, 1, RB)
    x16 = jnp.pad(x, ((0, 0), (0, 6)))
    x_split = jnp.stack([x16[:, :8], x16[:, 8:]])

    params = [(W1_0, b1_0, W2_0, b2_0, gamma_0, beta_0),
              (W1_1, b1_1, W2_1, b2_1, gamma_1, beta_1),
              (W1_2, b1_2, W2_2, b2_2, gamma_2, beta_2)]

    z_split = None
    node_rep = None
    for i, (w1, b1, w2, b2, gamma, beta) in enumerate(params):
        b18 = jnp.broadcast_to(b1[None, :], (8, EMB))
        b28 = jnp.broadcast_to(b2[None, :], (8, EMB))
        if i == 0:
            agg = _segment_sum(src_ab, dst_r, x_split)
            t, stats = _mlp0(x, agg, w1, b18, w2, b28)
        else:
            agg = _segment_sum(src_ab, dst_r, z_split)
            t, stats = _mlp(z_split, agg, w1, b18, w2, b28)
        scale8, shift8 = _bn_coeffs(stats, gamma, beta)
        if i < 2:
            z_split = _norm(t, scale8, shift8)
        else:
            node_rep, gsum, gcnt = _norm_pool(t, scale8, shift8, batch3)
    counts = gcnt[0]
    graph_rep = gsum / jnp.maximum(counts, 1.0)[:, None]
    return node_rep, graph_rep
